# Initial kernel scaffold; baseline (speedup 1.0000x reference)
import jax
import jax.numpy as jnp
from jax.experimental import pallas as pl


def kernel(images, patch_indices):
    raise NotImplementedError("write your pallas kernel here")


# speedup vs baseline: 1.1451x; 1.1451x over previous
import functools

import jax
import jax.numpy as jnp
from jax import lax
from jax.experimental import pallas as pl
from jax.experimental.pallas import tpu as pltpu
from jax.experimental.pallas import tpu_sc as plsc

PATCH = 16
LANES = 16
CHUNK_ROWS = 128


@functools.lru_cache(maxsize=None)
def _build_sc_call(N, H, W, K):
    WC = W // PATCH
    rows_per_img = H * WC
    KR = K * PATCH
    n_chunks = (KR + CHUNK_ROWS - 1) // CHUNK_ROWS
    tail_rows = KR - (n_chunks - 1) * CHUNK_ROWS
    n_groups = (n_chunks * CHUNK_ROWS + 255) // 256
    idx_rows = 2 * n_groups
    pi_pad = n_groups * LANES

    info = plsc.get_sparse_core_info()
    n_workers = info.num_cores * info.num_subcores
    batches_per_w = (N + n_workers - 1) // n_workers

    mesh = plsc.VectorSubcoreMesh(core_axis_name="c", subcore_axis_name="s")

    @functools.partial(
        pl.kernel,
        mesh=mesh,
        compiler_params=pltpu.CompilerParams(use_tc_tiling_on_sc=False),
        out_type=jax.ShapeDtypeStruct((N * KR, LANES), jnp.float32),
        scratch_types=[
            pltpu.VMEM((pi_pad,), jnp.int32),
            pltpu.VMEM((idx_rows, 128), jnp.int32),
            pltpu.VMEM((CHUNK_ROWS, LANES), jnp.float32),
            pltpu.VMEM((CHUNK_ROWS, LANES), jnp.float32),
            pltpu.SemaphoreType.DMA,
            pltpu.SemaphoreType.DMA,
        ],
    )
    def sc_kernel(tab_hbm, pidx_hbm, out_hbm, pi_v, idxbuf, buf0, buf1, sem0, sem1):
        wid = lax.axis_index("s") * info.num_cores + lax.axis_index("c")
        ii = lax.iota(jnp.int32, LANES)

        for g in range(K // LANES, n_groups):
            pi_v[pl.ds(g * LANES, LANES)] = jnp.zeros((LANES,), jnp.int32)
        pltpu.sync_copy(pidx_hbm, pi_v.at[pl.ds(0, K)])

        b0 = wid * batches_per_w
        boff0 = b0 * rows_per_img

        iiwc = ii * WC

        def build_group(g, carry):
            pi = pi_v[pl.ds(g * LANES, LANES)]
            base = (pi >> 5) * (PATCH * WC) + (pi & (WC - 1)) + boff0
            for j in range(LANES):
                bj = base.at[jnp.full((LANES,), j, jnp.int32)].get(
                    mode="promise_in_bounds")
                m = g * LANES + j
                idxbuf[m >> 3, pl.ds((m & 7) * PATCH, PATCH)] = bj + iiwc
            return carry

        lax.fori_loop(0, n_groups, build_group, 0)

        def shift_row(r, carry):
            for l in range(128 // LANES):
                sl = (r, pl.ds(l * LANES, LANES))
                idxbuf[sl] = idxbuf[sl] + rows_per_img
            return carry

        def chunk_pair(obase):
            def body(t, carry):
                c0 = t * 2
                c1 = c0 + 1
                cp0 = pltpu.async_copy(tab_hbm.at[idxbuf.at[c0]], buf0, sem0)
                cp1 = pltpu.async_copy(tab_hbm.at[idxbuf.at[c1]], buf1, sem1)
                cp0.wait()
                pltpu.sync_copy(
                    buf0, out_hbm.at[pl.ds(obase + c0 * CHUNK_ROWS, CHUNK_ROWS)]
                )
                cp1.wait()
                pltpu.sync_copy(
                    buf1, out_hbm.at[pl.ds(obase + c1 * CHUNK_ROWS, CHUNK_ROWS)]
                )
                return carry
            return body

        n_pairs = (n_chunks - 1) // 2
        for j in range(batches_per_w):
            if j > 0:
                lax.fori_loop(0, idx_rows, shift_row, 0)
            obase = (b0 + j) * KR
            lax.fori_loop(0, n_pairs, chunk_pair(obase), 0)
            for c in range(2 * n_pairs, n_chunks):
                nrows = tail_rows if c == n_chunks - 1 else CHUNK_ROWS
                cp = pltpu.async_copy(tab_hbm.at[idxbuf.at[c]], buf0, sem0)
                cp.wait()
                pltpu.sync_copy(
                    buf0.at[pl.ds(0, nrows)],
                    out_hbm.at[pl.ds(obase + c * CHUNK_ROWS, nrows)],
                )

    return sc_kernel


def kernel(images, patch_indices):
    N, H, W = images.shape
    K = patch_indices.shape[0]
    tab = images.reshape(N * H * (W // PATCH), PATCH)
    sc = _build_sc_call(N, H, W, K)
    out = sc(tab, patch_indices.astype(jnp.int32))
    return out.reshape(N, K, PATCH * PATCH)
